# baseline (speedup 1.0000x reference)
import functools

import jax
import jax.numpy as jnp
from jax import lax
from jax.experimental import pallas as pl
from jax.experimental.pallas import tpu as pltpu
from jax.experimental.pallas import tpu_sc as plsc

HID = 128
EPS = 1e-12
CHUNK = 128


@functools.lru_cache(maxsize=None)
def _make_sc_gather(n_tokens: int):
    info = plsc.get_sparse_core_info()
    nc, ns = info.num_cores, info.num_subcores
    nw = nc * ns
    per_w = n_tokens // nw
    iters = per_w // CHUNK
    mesh = plsc.VectorSubcoreMesh(core_axis_name="c", subcore_axis_name="s")

    nbuf = 4
    ngroups = iters // nbuf

    @functools.partial(
        pl.kernel,
        out_type=jax.ShapeDtypeStruct((n_tokens, HID), jnp.float32),
        mesh=mesh,
        scratch_types=(
            [pltpu.VMEM((per_w,), jnp.int32)]
            + [pltpu.VMEM((CHUNK, HID), jnp.float32)] * nbuf
            + [pltpu.SemaphoreType.DMA] * (2 * nbuf)
        ),
    )
    def gather(table_hbm, ids_hbm, out_hbm, idx_v, *bufs_and_sems):
        bufs = bufs_and_sems[:nbuf]
        gsems = bufs_and_sems[nbuf:2 * nbuf]
        wsems = bufs_and_sems[2 * nbuf:]
        wid = lax.axis_index("s") * nc + lax.axis_index("c")
        base = wid * per_w
        pltpu.sync_copy(ids_hbm.at[pl.ds(base, per_w)], idx_v)

        def g(i, q):
            return pltpu.make_async_copy(
                table_hbm.at[idx_v.at[pl.ds(i * CHUNK, CHUNK)]],
                bufs[q], gsems[q])

        def w(i, q):
            return pltpu.make_async_copy(
                bufs[q], out_hbm.at[pl.ds(base + i * CHUNK, CHUNK)],
                wsems[q])

        for q in range(nbuf):
            g(q, q).start()

        def body(j, carry):
            i0 = nbuf * j
            for q in range(nbuf):
                i = i0 + q
                g(i, q).wait()
                w(i, q).start()

                @pl.when(j < ngroups - 1)
                def _(i=i, q=q):
                    w(i, q).wait()
                    g(i + nbuf, q).start()

            return carry

        lax.fori_loop(0, ngroups, body, 0)
        for q in range(nbuf):
            w(iters - nbuf + q, q).wait()

    return gather


def _ln_body(gath_ref, code_ref, pos_ref, ctab_ref, gamma_ref, beta_ref,
             out_ref):
    r, s = gath_ref.shape[0], gath_ref.shape[1]
    j = jnp.full((HID, HID), 1.0 / HID, dtype=jnp.float32)
    for q in range(r):
        code = code_ref[q, 0, :][:, None]
        oh = (code == lax.broadcasted_iota(jnp.int32, (s, 32), 1)
              ).astype(jnp.float32)
        e = (gath_ref[q] + pos_ref[...]
             + jnp.dot(oh, ctab_ref[...], preferred_element_type=jnp.float32))
        m1 = jnp.dot(e, j, preferred_element_type=jnp.float32)
        m2 = jnp.dot(e * e, j, preferred_element_type=jnp.float32)
        out_ref[q] = ((e - m1) * lax.rsqrt(m2 - m1 * m1 + EPS)
                      * gamma_ref[...] + beta_ref[...])


def _ln_body_chain(dst_ref, gath_ref, code_ref, pos_ref, ctab_ref, gamma_ref,
                   beta_ref, out_ref):
    del dst_ref
    _ln_body(gath_ref, code_ref, pos_ref, ctab_ref, gamma_ref, beta_ref,
             out_ref)


def kernel(input_ids, token_type_ids, task_type_ids, word_emb, pos_emb,
           tok_emb, task_emb, gamma, beta):
    b, s = input_ids.shape
    slices = (128, 256, 256, 256, 128)
    rows = 32
    ids = input_ids.reshape(b * s).astype(jnp.int32)
    code3 = (token_type_ids.astype(jnp.int32)
             + 2 * task_type_ids.astype(jnp.int32)).reshape(b, 1, s)
    ar = jnp.arange(32)
    ctab = tok_emb[ar % 2] + task_emb[ar // 2]
    gamma2 = gamma.reshape(1, HID)
    beta2 = beta.reshape(1, HID)

    offs = [0]
    for w in slices:
        offs.append(offs[-1] + w)
    gaths = [
        _make_sc_gather(w * s)(
            word_emb, lax.slice(ids, (o * s,), ((o + w) * s,))
        ).reshape(w, s, HID)
        for o, w in zip(offs, slices)
    ]

    in_specs = [
        pl.BlockSpec((rows, s, HID), lambda i: (i, 0, 0)),
        pl.BlockSpec((rows, 1, s), lambda i: (i, 0, 0)),
        pl.BlockSpec((s, HID), lambda i: (0, 0)),
        pl.BlockSpec((32, HID), lambda i: (0, 0)),
        pl.BlockSpec((1, HID), lambda i: (0, 0)),
        pl.BlockSpec((1, HID), lambda i: (0, 0)),
    ]
    out_shape = jax.ShapeDtypeStruct((b, s, HID), jnp.float32)
    cparams = pltpu.CompilerParams(dimension_semantics=("arbitrary",))

    out = None
    for k, (o, w) in enumerate(zip(offs, slices)):
        code_k = lax.slice_in_dim(code3, o, o + w, axis=0)
        args = (gaths[k], code_k, pos_emb, ctab, gamma2, beta2)
        base_blk = o // rows
        out_spec = pl.BlockSpec(
            (rows, s, HID), lambda i, bb=base_blk: (bb + i, 0, 0))
        if k == 0:
            out = pl.pallas_call(
                _ln_body, grid=(w // rows,), in_specs=in_specs,
                out_specs=out_spec, out_shape=out_shape,
                compiler_params=cparams)(*args)
        else:
            out = pl.pallas_call(
                _ln_body_chain, grid=(w // rows,),
                in_specs=[pl.BlockSpec(memory_space=pl.ANY)] + in_specs,
                out_specs=out_spec, out_shape=out_shape,
                input_output_aliases={0: 0},
                compiler_params=cparams)(out, *args)
    return out


# speedup vs baseline: 1.0123x; 1.0123x over previous
import functools

import jax
import jax.numpy as jnp
from jax import lax
from jax.experimental import pallas as pl
from jax.experimental.pallas import tpu as pltpu
from jax.experimental.pallas import tpu_sc as plsc

HID = 128
EPS = 1e-12
CHUNK = 128


@functools.lru_cache(maxsize=None)
def _make_sc_gather(n_tokens: int):
    info = plsc.get_sparse_core_info()
    nc, ns = info.num_cores, info.num_subcores
    nw = nc * ns
    per_w = n_tokens // nw
    iters = per_w // CHUNK
    mesh = plsc.VectorSubcoreMesh(core_axis_name="c", subcore_axis_name="s")

    nbuf = 4
    ngroups = iters // nbuf

    @functools.partial(
        pl.kernel,
        out_type=jax.ShapeDtypeStruct((n_tokens, HID), jnp.float32),
        mesh=mesh,
        scratch_types=(
            [pltpu.VMEM((per_w,), jnp.int32)]
            + [pltpu.VMEM((CHUNK, HID), jnp.float32)] * nbuf
            + [pltpu.SemaphoreType.DMA] * (2 * nbuf)
        ),
    )
    def gather(table_hbm, ids_hbm, out_hbm, idx_v, *bufs_and_sems):
        bufs = bufs_and_sems[:nbuf]
        gsems = bufs_and_sems[nbuf:2 * nbuf]
        wsems = bufs_and_sems[2 * nbuf:]
        wid = lax.axis_index("s") * nc + lax.axis_index("c")
        base = wid * per_w
        pltpu.sync_copy(ids_hbm.at[pl.ds(base, per_w)], idx_v)

        def g(i, q):
            return pltpu.make_async_copy(
                table_hbm.at[idx_v.at[pl.ds(i * CHUNK, CHUNK)]],
                bufs[q], gsems[q])

        def w(i, q):
            return pltpu.make_async_copy(
                bufs[q], out_hbm.at[pl.ds(base + i * CHUNK, CHUNK)],
                wsems[q])

        for q in range(nbuf):
            g(q, q).start()

        def body(j, carry):
            i0 = nbuf * j
            for q in range(nbuf):
                i = i0 + q
                g(i, q).wait()
                w(i, q).start()

                @pl.when(j < ngroups - 1)
                def _(i=i, q=q):
                    w(i, q).wait()
                    g(i + nbuf, q).start()

            return carry

        lax.fori_loop(0, ngroups, body, 0)
        for q in range(nbuf):
            w(iters - nbuf + q, q).wait()

    return gather


def _ln_body(gath_ref, code_ref, pos_ref, ctab_ref, gamma_ref, beta_ref,
             out_ref):
    r, s = gath_ref.shape[0], gath_ref.shape[1]
    j = jnp.full((HID, HID), 1.0 / HID, dtype=jnp.float32)
    for q in range(r):
        code = code_ref[q, 0, :][:, None]
        oh = (code == lax.broadcasted_iota(jnp.int32, (s, 32), 1)
              ).astype(jnp.float32)
        e = (gath_ref[q] + pos_ref[...]
             + jnp.dot(oh, ctab_ref[...], preferred_element_type=jnp.float32))
        m1 = jnp.dot(e, j, preferred_element_type=jnp.float32)
        m2 = jnp.dot(e * e, j, preferred_element_type=jnp.float32)
        out_ref[q] = ((e - m1) * lax.rsqrt(m2 - m1 * m1 + EPS)
                      * gamma_ref[...] + beta_ref[...])


def _ln_body_chain(dst_ref, gath_ref, code_ref, pos_ref, ctab_ref, gamma_ref,
                   beta_ref, out_ref):
    del dst_ref
    _ln_body(gath_ref, code_ref, pos_ref, ctab_ref, gamma_ref, beta_ref,
             out_ref)


def kernel(input_ids, token_type_ids, task_type_ids, word_emb, pos_emb,
           tok_emb, task_emb, gamma, beta):
    b, s = input_ids.shape
    slices = (256, 256, 256, 256)
    rows = 32
    ids = input_ids.reshape(b * s).astype(jnp.int32)
    code3 = (token_type_ids.astype(jnp.int32)
             + 2 * task_type_ids.astype(jnp.int32)).reshape(b, 1, s)
    ar = jnp.arange(32)
    ctab = tok_emb[ar % 2] + task_emb[ar // 2]
    gamma2 = gamma.reshape(1, HID)
    beta2 = beta.reshape(1, HID)

    offs = [0]
    for w in slices:
        offs.append(offs[-1] + w)
    gaths = [
        _make_sc_gather(w * s)(
            word_emb, lax.slice(ids, (o * s,), ((o + w) * s,))
        ).reshape(w, s, HID)
        for o, w in zip(offs, slices)
    ]

    in_specs = [
        pl.BlockSpec((rows, s, HID), lambda i: (i, 0, 0)),
        pl.BlockSpec((rows, 1, s), lambda i: (i, 0, 0)),
        pl.BlockSpec((s, HID), lambda i: (0, 0)),
        pl.BlockSpec((32, HID), lambda i: (0, 0)),
        pl.BlockSpec((1, HID), lambda i: (0, 0)),
        pl.BlockSpec((1, HID), lambda i: (0, 0)),
    ]
    out_shape = jax.ShapeDtypeStruct((b, s, HID), jnp.float32)
    cparams = pltpu.CompilerParams(dimension_semantics=("arbitrary",))

    out = None
    for k, (o, w) in enumerate(zip(offs, slices)):
        code_k = lax.slice_in_dim(code3, o, o + w, axis=0)
        args = (gaths[k], code_k, pos_emb, ctab, gamma2, beta2)
        base_blk = o // rows
        out_spec = pl.BlockSpec(
            (rows, s, HID), lambda i, bb=base_blk: (bb + i, 0, 0))
        if k == 0:
            out = pl.pallas_call(
                _ln_body, grid=(w // rows,), in_specs=in_specs,
                out_specs=out_spec, out_shape=out_shape,
                compiler_params=cparams)(*args)
        else:
            out = pl.pallas_call(
                _ln_body_chain, grid=(w // rows,),
                in_specs=[pl.BlockSpec(memory_space=pl.ANY)] + in_specs,
                out_specs=out_spec, out_shape=out_shape,
                input_output_aliases={0: 0},
                compiler_params=cparams)(out, *args)
    return out
